# baseline (speedup 1.0000x reference)
import functools

import jax
import jax.numpy as jnp
from jax import lax
from jax.experimental import pallas as pl
from jax.experimental.pallas import tpu as pltpu
from jax.experimental.pallas import tpu_sc as plsc

C = 32
NC, NS = 2, 16
NW = NC * NS
GBLK = 128
BLK = 256
P = 4
EPS = 1e-5
TC_BLK = 1000


def _tc_ln_body(t_ref, g_ref, b_ref, o_ref):
    x = t_ref[...]
    mean = jnp.mean(x, axis=1, keepdims=True)
    var = jnp.mean((x - mean) ** 2, axis=1, keepdims=True)
    normed = (x - mean) / jnp.sqrt(var + EPS)
    o_ref[...] = normed * g_ref[...] + b_ref[...]


def _normalize_table(table, gamma, beta):
    v = table.shape[0]
    assert v % TC_BLK == 0
    return pl.pallas_call(
        _tc_ln_body,
        grid=(v // TC_BLK,),
        in_specs=[
            pl.BlockSpec((TC_BLK, C), lambda i: (i, 0)),
            pl.BlockSpec((1, C), lambda i: (0, 0)),
            pl.BlockSpec((1, C), lambda i: (0, 0)),
        ],
        out_specs=pl.BlockSpec((TC_BLK, C), lambda i: (i, 0)),
        out_shape=jax.ShapeDtypeStruct((v, C), jnp.float32),
    )(table, gamma.reshape(1, C), beta.reshape(1, C))


def _make_sc_gather(n):
    assert n % (NW * BLK) == 0
    blocks_per_w = n // (NW * BLK)
    assert blocks_per_w >= P
    mesh = plsc.VectorSubcoreMesh(core_axis_name="c", subcore_axis_name="s")

    @functools.partial(
        pl.kernel,
        out_type=jax.ShapeDtypeStruct((n, C), jnp.float32),
        mesh=mesh,
        compiler_params=pltpu.CompilerParams(
            needs_layout_passes=False, use_tc_tiling_on_sc=False),
        scratch_types=[
            pltpu.VMEM((P, 2, GBLK), jnp.int32),
            pltpu.VMEM((P, BLK, C), jnp.float32),
        ] + [pltpu.SemaphoreType.DMA] * (3 * P),
    )
    def sc_gather(x_hbm, table_hbm, out_hbm, idx_v, rows_v, *sems):
        gsem = sems[0:P]
        isem = sems[P:2 * P]
        wsem = sems[2 * P:3 * P]
        wid = lax.axis_index("s") * NC + lax.axis_index("c")
        base = wid * (blocks_per_w * BLK)

        def launch_gathers(p):
            for h in range(2):
                pltpu.async_copy(
                    table_hbm.at[idx_v.at[p, h]],
                    rows_v.at[p, pl.ds(h * GBLK, GBLK)], gsem[p])

        def wait_gathers(p):
            for h in range(2):
                pltpu.make_async_copy(
                    table_hbm.at[idx_v.at[p, h]],
                    rows_v.at[p, pl.ds(h * GBLK, GBLK)], gsem[p]).wait()

        for p in range(2):
            for h in range(2):
                pltpu.sync_copy(
                    x_hbm.at[pl.ds(base + (p * 2 + h) * GBLK, GBLK)],
                    idx_v.at[p, h])
            launch_gathers(p)

        def body(i, carry):
            row0 = base + i * BLK
            for p in range(P):

                @pl.when(i % P == p)
                def _():
                    p2 = (p + 2) % P

                    @pl.when(i + 2 < blocks_per_w)
                    def _():
                        for h in range(2):
                            pltpu.async_copy(
                                x_hbm.at[pl.ds(row0 + 2 * BLK + h * GBLK,
                                               GBLK)],
                                idx_v.at[p2, h], isem[p2])

                    wait_gathers(p)
                    pltpu.async_copy(rows_v.at[p],
                                     out_hbm.at[pl.ds(row0, BLK)], wsem[p])

                    @pl.when(i + 2 < blocks_per_w)
                    def _():
                        @pl.when(i >= 2)
                        def _():
                            pltpu.make_async_copy(
                                rows_v.at[p2],
                                out_hbm.at[pl.ds(row0 - 2 * BLK, BLK)],
                                wsem[p2]).wait()
                        for h in range(2):
                            pltpu.make_async_copy(
                                x_hbm.at[pl.ds(row0 + 2 * BLK + h * GBLK,
                                               GBLK)],
                                idx_v.at[p2, h], isem[p2]).wait()
                        launch_gathers(p2)

            return carry

        lax.fori_loop(0, blocks_per_w, body, 0)

        for k in range(4):
            i_last = blocks_per_w - 1 - k
            pltpu.make_async_copy(
                rows_v.at[i_last % P],
                out_hbm.at[pl.ds(base + i_last * BLK, BLK)],
                wsem[i_last % P]).wait()

    return sc_gather


def kernel(x, table, gamma, beta):
    b, l = x.shape
    n = b * l
    xf = x.reshape(n).astype(jnp.int32)
    table_n = _normalize_table(table, gamma, beta)
    out = _make_sc_gather(n)(xf, table_n)
    one = 1.0 + 0.0 * gamma[0]
    return out.reshape(b, l, C) * one


# speedup vs baseline: 1.4875x; 1.4875x over previous
import functools

import jax
import jax.numpy as jnp
from jax import lax
from jax.experimental import pallas as pl
from jax.experimental.pallas import tpu as pltpu
from jax.experimental.pallas import tpu_sc as plsc

C = 32
NC, NS = 2, 16
NW = NC * NS
GBLK = 128
BLK = 256
P = 4
EPS = 1e-5
TC_BLK = 1000


def _tc_ln_body(t_ref, g_ref, b_ref, o_ref):
    x = t_ref[...]
    mean = jnp.mean(x, axis=1, keepdims=True)
    var = jnp.mean((x - mean) ** 2, axis=1, keepdims=True)
    normed = (x - mean) / jnp.sqrt(var + EPS)
    o_ref[...] = normed * g_ref[...] + b_ref[...]


def _normalize_table(table, gamma, beta):
    v = table.shape[0]
    assert v % TC_BLK == 0
    return pl.pallas_call(
        _tc_ln_body,
        grid=(v // TC_BLK,),
        in_specs=[
            pl.BlockSpec((TC_BLK, C), lambda i: (i, 0)),
            pl.BlockSpec((1, C), lambda i: (0, 0)),
            pl.BlockSpec((1, C), lambda i: (0, 0)),
        ],
        out_specs=pl.BlockSpec((TC_BLK, C), lambda i: (i, 0)),
        out_shape=jax.ShapeDtypeStruct((v, C), jnp.float32),
    )(table, gamma.reshape(1, C), beta.reshape(1, C))


def _make_sc_gather(n):
    assert n % (NW * BLK) == 0
    blocks_per_w = n // (NW * BLK)
    assert blocks_per_w >= P
    mesh = plsc.VectorSubcoreMesh(core_axis_name="c", subcore_axis_name="s")

    @functools.partial(
        pl.kernel,
        out_type=jax.ShapeDtypeStruct((n, C), jnp.float32),
        mesh=mesh,
        compiler_params=pltpu.CompilerParams(
            needs_layout_passes=False, use_tc_tiling_on_sc=False),
        scratch_types=[
            pltpu.VMEM((P, 2, GBLK), jnp.int32),
            pltpu.VMEM((P, BLK, C), jnp.float32),
        ] + [pltpu.SemaphoreType.DMA] * (3 * P),
    )
    def sc_gather(x_hbm, table_hbm, out_hbm, idx_v, rows_v, *sems):
        gsem = sems[0:P]
        isem = sems[P:2 * P]
        wsem = sems[2 * P:3 * P]
        wid = lax.axis_index("s") * NC + lax.axis_index("c")
        base = wid * (blocks_per_w * BLK)

        def launch_gathers(p):
            for h in range(2):
                pltpu.async_copy(
                    table_hbm.at[idx_v.at[p, h]],
                    rows_v.at[p, pl.ds(h * GBLK, GBLK)], gsem[p])

        def wait_gathers(p):
            for h in range(2):
                pltpu.make_async_copy(
                    table_hbm.at[idx_v.at[p, h]],
                    rows_v.at[p, pl.ds(h * GBLK, GBLK)], gsem[p]).wait()

        for p in range(2):
            for h in range(2):
                pltpu.sync_copy(
                    x_hbm.at[pl.ds(base + (p * 2 + h) * GBLK, GBLK)],
                    idx_v.at[p, h])
            launch_gathers(p)

        def body(i, carry):
            row0 = base + i * BLK
            for p in range(P):

                @pl.when(i % P == p)
                def _():
                    p2 = (p + 2) % P

                    @pl.when(i + 2 < blocks_per_w)
                    def _():
                        for h in range(2):
                            pltpu.async_copy(
                                x_hbm.at[pl.ds(row0 + 2 * BLK + h * GBLK,
                                               GBLK)],
                                idx_v.at[p2, h], isem[p2])

                    wait_gathers(p)
                    pltpu.async_copy(rows_v.at[p],
                                     out_hbm.at[pl.ds(row0, BLK)], wsem[p])

                    @pl.when(i + 2 < blocks_per_w)
                    def _():
                        @pl.when(i >= 2)
                        def _():
                            pltpu.make_async_copy(
                                rows_v.at[p2],
                                out_hbm.at[pl.ds(row0 - 2 * BLK, BLK)],
                                wsem[p2]).wait()
                        for h in range(2):
                            pltpu.make_async_copy(
                                x_hbm.at[pl.ds(row0 + 2 * BLK + h * GBLK,
                                               GBLK)],
                                idx_v.at[p2, h], isem[p2]).wait()
                        launch_gathers(p2)

            return carry

        lax.fori_loop(0, blocks_per_w, body, 0)

        for k in range(4):
            i_last = blocks_per_w - 1 - k
            pltpu.make_async_copy(
                rows_v.at[i_last % P],
                out_hbm.at[pl.ds(base + i_last * BLK, BLK)],
                wsem[i_last % P]).wait()

    return sc_gather


def kernel(x, table, gamma, beta):
    b, l = x.shape
    n = b * l
    xf = x.reshape(n).astype(jnp.int32)
    table_n = _normalize_table(table, gamma, beta)
    out = _make_sc_gather(n)(xf, table_n)
    return out.reshape(b, l, C)
